# Initial kernel scaffold; baseline (speedup 1.0000x reference)
import jax
import jax.numpy as jnp
from jax.experimental import pallas as pl


def kernel(u_idx, r_idx, v_idx, drug_emb, rel_emb, target_emb):
    raise NotImplementedError("write your pallas kernel here")


# speedup vs baseline: 2.3199x; 2.3199x over previous
import functools

import jax
import jax.numpy as jnp
from jax import lax
from jax.experimental import pallas as pl
from jax.experimental.pallas import tpu as pltpu
from jax.experimental.pallas import tpu_sc as plsc

_MARGIN = 12.0
_DIM = 128
_DPAD = 1024
_RPAD = 256
_B = 4096
_NNEG = 32

_OFF_RT = _DPAD * _DPAD
_OFF_A = _OFF_RT + _RPAD * _DPAD

_NC, _NS = 2, 16
_NW = _NC * _NS
_NIDX = _B * _NNEG * 2 + _B
_ROWS_PW = 72
_IDX_ROWS = _ROWS_PW * _NW


def _prep_body(drug_ref, rel_ref, tgt_ref, u_ref, r_ref, v_ref,
               dt_ref, rt_ref, a_ref, idt_ref, irt_ref, ia_ref):
    d = drug_ref[...]
    r = rel_ref[...]
    t = tgt_ref[...]
    dn = jnp.sum(d * d, axis=1, keepdims=True)
    rn = jnp.sum(r * r, axis=1, keepdims=True)
    tn = jnp.sum(t * t, axis=1, keepdims=True)
    ones_d = jnp.ones((_DPAD, 1), jnp.float32)
    ones_r = jnp.ones((_RPAD, 1), jnp.float32)
    d1 = jnp.concatenate([d, ones_d], axis=1)
    t1 = jnp.concatenate([-2.0 * t, tn], axis=1)
    dt_ref[...] = lax.dot_general(
        d1, t1, (((1,), (1,)), ((), ())),
        preferred_element_type=jnp.float32, precision=lax.Precision.HIGHEST)
    rt_ref[...] = lax.dot_general(
        -2.0 * r, t, (((1,), (1,)), ((), ())),
        preferred_element_type=jnp.float32, precision=lax.Precision.HIGHEST)
    d2 = jnp.concatenate([d, dn, ones_d], axis=1)
    r2 = jnp.concatenate([2.0 * r, ones_r, rn], axis=1)
    a_ref[...] = lax.dot_general(
        d2, r2, (((1,), (1,)), ((), ())),
        preferred_element_type=jnp.float32, precision=lax.Precision.HIGHEST)
    u = u_ref[...]
    ri = r_ref[...]
    v = v_ref[...]
    idt_ref[...] = u * _DPAD + v
    irt_ref[...] = _OFF_RT + ri * _DPAD + v
    ia_ref[...] = _OFF_A + u * _RPAD + ri


def _fin_body(gdt_ref, grt_ref, ga_ref, out_ref):
    s = gdt_ref[...] + grt_ref[...] + ga_ref[...]
    out_ref[...] = _MARGIN - jnp.sqrt(jnp.maximum(s, 0.0))


@functools.partial(
    pl.kernel,
    mesh=plsc.VectorSubcoreMesh(core_axis_name="c", subcore_axis_name="s"),
    out_type=jax.ShapeDtypeStruct((_IDX_ROWS, 128), jnp.float32),
    scratch_types=[
        pltpu.VMEM((_ROWS_PW, 128), jnp.int32),
        pltpu.VMEM((_ROWS_PW, 128), jnp.float32),
        pltpu.SemaphoreType.DMA,
    ],
)
def _sc_gather(tbl_hbm, idx_hbm, out_hbm, idx_v, g_v, sem):
    wid = lax.axis_index("s") * _NC + lax.axis_index("c")
    base = wid * _ROWS_PW
    pltpu.sync_copy(idx_hbm.at[pl.ds(base, _ROWS_PW)], idx_v)

    def fire(i, carry):
        pltpu.make_async_copy(tbl_hbm.at[idx_v.at[i]], g_v.at[i], sem).start()
        return carry

    lax.fori_loop(0, _ROWS_PW, fire, 0)
    pltpu.make_async_copy(out_hbm.at[pl.ds(base, _ROWS_PW)], g_v, sem).wait()
    pltpu.sync_copy(g_v, out_hbm.at[pl.ds(base, _ROWS_PW)])


def kernel(u_idx, r_idx, v_idx, drug_emb, rel_emb, target_emb):
    u = u_idx.astype(jnp.int32).reshape(_B, 1)
    r = r_idx.astype(jnp.int32).reshape(_B, 1)
    v = v_idx.astype(jnp.int32)
    drug = jnp.pad(drug_emb, ((0, _DPAD - drug_emb.shape[0]), (0, 0)))
    rel = jnp.pad(rel_emb, ((0, _RPAD - rel_emb.shape[0]), (0, 0)))
    tgt = jnp.pad(target_emb, ((0, _DPAD - target_emb.shape[0]), (0, 0)))

    dt, rt, a, idt, irt, ia = pl.pallas_call(
        _prep_body,
        out_shape=[
            jax.ShapeDtypeStruct((_DPAD, _DPAD), jnp.float32),
            jax.ShapeDtypeStruct((_RPAD, _DPAD), jnp.float32),
            jax.ShapeDtypeStruct((_DPAD, _RPAD), jnp.float32),
            jax.ShapeDtypeStruct((_B, _NNEG), jnp.int32),
            jax.ShapeDtypeStruct((_B, _NNEG), jnp.int32),
            jax.ShapeDtypeStruct((_B, 1), jnp.int32),
        ],
    )(drug, rel, tgt, u, r, v)

    tbl = jnp.concatenate([dt.reshape(-1), rt.reshape(-1), a.reshape(-1)])
    idx = jnp.concatenate([idt.reshape(-1), irt.reshape(-1), ia.reshape(-1)])
    idx = jnp.pad(idx, (0, _IDX_ROWS * 128 - _NIDX))
    g = _sc_gather(tbl, idx.reshape(_IDX_ROWS, 128)).reshape(-1)

    gdt = g[: _B * _NNEG].reshape(_B, _NNEG)
    grt = g[_B * _NNEG: 2 * _B * _NNEG].reshape(_B, _NNEG)
    ga = g[2 * _B * _NNEG: _NIDX].reshape(_B, 1)

    out = pl.pallas_call(
        _fin_body,
        out_shape=jax.ShapeDtypeStruct((_B, _NNEG), jnp.float32),
    )(gdt, grt, ga)
    return out
